# baseline (speedup 1.0000x reference)
import functools
import jax
import jax.numpy as jnp
from jax import lax
from jax.experimental import pallas as pl
from jax.experimental.pallas import tpu as pltpu
from jax.experimental.pallas import tpu_sc as plsc

_D = 4096
_H = 32
_L = 4
_N = 64
_THR = 0.3
_LN2 = 0.6931471805599453
_CH = 1024
_NC = _D // _CH

_OFF_AQ = 0
_OFF_GQ = _L * _N
_OFF_SA2 = 2 * _L * _N
_OFF_SG2 = 3 * _L * _N
_OFF_MT = 4 * _L * _N
_BUF = _OFF_MT + (_L - 1) * _N * _N
_OUT = (_L + 1) * _N


def _precompute_body(q_hbm, ops_hbm, wq_hbm, bq_ref, wo_hbm, bo_ref, out_ref,
                     q_v, ops_v, wq_v, wa_v, wg_v, sems):
    cps = []
    for c in range(_NC):
        d = pl.ds(c * _CH, _CH)
        group = [
            pltpu.make_async_copy(ops_hbm.at[:, d], ops_v.at[:, d], sems.at[c, 0]),
            pltpu.make_async_copy(wo_hbm.at[:, d], wa_v.at[:, d], sems.at[c, 1]),
            pltpu.make_async_copy(wo_hbm.at[:, pl.ds(_D + c * _CH, _CH)], wg_v.at[:, d],
                                  sems.at[c, 2]),
            pltpu.make_async_copy(wq_hbm.at[:, d], wq_v.at[:, d], sems.at[c, 3]),
        ]
        cps.append(group)
    qcp = pltpu.make_async_copy(q_hbm, q_v, sems.at[_NC, 0])
    for group in cps:
        for cp in group:
            cp.start()
    qcp.start()
    qcp.wait()

    dn = (((1,), (1,)), ((), ()))
    qproj = jnp.zeros((1, _L * _H), dtype=jnp.float32)
    A = jnp.zeros((_N, _L * _H), dtype=jnp.float32)
    G = jnp.zeros((_N, _L * _H), dtype=jnp.float32)
    for c in range(_NC):
        d = pl.ds(c * _CH, _CH)
        for cp in cps[c]:
            cp.wait()
        opsc = ops_v[:, d]
        A = A + jax.lax.dot_general(opsc, wa_v[:, d], dn, preferred_element_type=jnp.float32)
        G = G + jax.lax.dot_general(opsc, wg_v[:, d], dn, preferred_element_type=jnp.float32)
        qproj = qproj + jax.lax.dot_general(q_v[:, d], wq_v[:, d], dn,
                                            preferred_element_type=jnp.float32)

    bq_all = bq_ref[...]
    bo_all = bo_ref[...]
    ones_h = jnp.ones((1, _H), dtype=jnp.float32)
    aq_rows, gq_rows, sa2_rows, sg2_rows, m_blocks = [], [], [], [], []
    for l in range(_L):
        qs = qproj[:, l * _H:(l + 1) * _H] + bq_all[l:l + 1, :]
        qn = qs / jnp.maximum(jnp.sqrt(jnp.sum(qs * qs)), 1e-12)
        Ap = A[:, l * _H:(l + 1) * _H] + bo_all[l:l + 1, :]
        Gl = G[:, l * _H:(l + 1) * _H]
        aq_rows.append(jax.lax.dot_general(qn, Ap, dn, preferred_element_type=jnp.float32))
        gq_rows.append(jax.lax.dot_general(qn, Gl, dn, preferred_element_type=jnp.float32))
        sa2_rows.append(jax.lax.dot_general(ones_h, Ap * Ap, dn, preferred_element_type=jnp.float32))
        sg2_rows.append(jax.lax.dot_general(ones_h, Gl * Gl, dn, preferred_element_type=jnp.float32))
        if l > 0:
            m_blocks.append(jax.lax.dot_general(Gl, Ap, dn, preferred_element_type=jnp.float32))

    out_ref[...] = jnp.concatenate(
        aq_rows + gq_rows + sa2_rows + sg2_rows + m_blocks, axis=0)


def _rsqrt_nr(x):
    bits = lax.bitcast_convert_type(x, jnp.int32)
    y = lax.bitcast_convert_type(jnp.int32(0x5F3759DF) - (bits >> 1), jnp.float32)
    for _ in range(3):
        y = y * (1.5 - 0.5 * x * y * y)
    return y


def _log_nr(x):
    bits = lax.bitcast_convert_type(x, jnp.int32)
    ex = ((bits >> 23) & jnp.int32(0xFF)) - 127
    mant = lax.bitcast_convert_type((bits & jnp.int32(0x007FFFFF)) | jnp.int32(0x3F800000),
                                    jnp.float32)
    y = ex.astype(jnp.float32) * _LN2 + (mant - 1.0)
    for _ in range(4):
        y = y + x * jnp.exp(-y) - 1.0
    return y


def _vgather(x, idx):
    dnums = lax.GatherDimensionNumbers(offset_dims=(), collapsed_slice_dims=(0,),
                                       start_index_map=(0,))
    return lax.gather(x, idx[:, None], dnums, slice_sizes=(1,),
                      mode=lax.GatherScatterMode.PROMISE_IN_BOUNDS)


def _sc_select_body(buf_hbm, out_hbm, buf_v, out_v):
    cid = lax.axis_index("c")
    sid = lax.axis_index("s")

    @pl.when(jnp.logical_and(cid == 0, sid == 0))
    def _():
        pltpu.sync_copy(buf_hbm, buf_v)

        iota = lax.iota(jnp.int32, 16)

        def red(v, op):
            for sh in (8, 4, 2, 1):
                v = op(v, _vgather(v, iota ^ sh))
            return v

        def add(a, b):
            return a + b

        onef = jnp.broadcast_to(jnp.float32(1.0), (16,))
        zerof = jnp.broadcast_to(jnp.float32(0.0), (16,))
        miss = jnp.broadcast_to(jnp.int32(192), (16,))
        fs = jnp.int32(0)
        for l in range(_L):
            sr = [buf_v[pl.ds(_OFF_AQ + l * _N + 16 * j, 16)] for j in range(4)]
            ss = [buf_v[pl.ds(_OFF_SA2 + l * _N + 16 * j, 16)] for j in range(4)]
            if l > 0:
                gq_f = jnp.broadcast_to(buf_v[pl.ds(_OFF_GQ + l * _N + fs, 16)][0], (16,))
                sg_f = jnp.broadcast_to(buf_v[pl.ds(_OFF_SG2 + l * _N + fs, 16)][0], (16,))
                mbase = _OFF_MT + (l - 1) * _N * _N + fs * _N
                for j in range(4):
                    mcol = buf_v[pl.ds(mbase + 16 * j, 16)]
                    ss[j] = ss[j] + 2.0 * mcol + sg_f
                    sr[j] = sr[j] + gq_f
            sc = [sr[j] * _rsqrt_nr(jnp.maximum(ss[j], 1e-24)) for j in range(4)]
            mx = red(jnp.maximum(jnp.maximum(sc[0], sc[1]),
                                 jnp.maximum(sc[2], sc[3])), jnp.maximum)
            e = [jnp.exp(sc[j] - mx) for j in range(4)]
            sv = red(e[0] + e[1] + e[2] + e[3], add)
            p = [e[j] / sv for j in range(4)]
            logs = _log_nr(sv)
            lp = [sc[j] - mx - logs for j in range(4)]
            maskb = [p[j] > _THR for j in range(4)]
            maskf = [jnp.where(maskb[j], onef, zerof) for j in range(4)]
            pmax = red(jnp.maximum(jnp.maximum(p[0], p[1]),
                                   jnp.maximum(p[2], p[3])), jnp.maximum)
            key = red(jnp.minimum(
                jnp.minimum(
                    jnp.where(maskb[0], iota,
                              jnp.where(p[0] == pmax, iota + 64, miss)),
                    jnp.where(maskb[1], iota + 16,
                              jnp.where(p[1] == pmax, iota + 80, miss))),
                jnp.minimum(
                    jnp.where(maskb[2], iota + 32,
                              jnp.where(p[2] == pmax, iota + 96, miss)),
                    jnp.where(maskb[3], iota + 48,
                              jnp.where(p[3] == pmax, iota + 112, miss)))),
                jnp.minimum)
            has_any = key < _N
            fidx = key & 63
            sel = [jnp.where(has_any, maskf[j],
                             jnp.where((iota + 16 * j) == fidx, onef, zerof))
                   for j in range(4)]
            llp = red(sel[0] * lp[0] + sel[1] * lp[1] + sel[2] * lp[2] + sel[3] * lp[3],
                      add)
            fs = fidx[0]
            for j in range(4):
                out_v[pl.ds(l * _N + 16 * j, 16)] = p[j]
            out_v[pl.ds(_L * _N + l * 16, 16)] = llp

        pltpu.sync_copy(out_v, out_hbm)


_sc_select_cache = []


def _get_sc_select():
    if not _sc_select_cache:
        _sc_select_cache.append(functools.partial(
            pl.kernel,
            out_type=jax.ShapeDtypeStruct((_OUT,), jnp.float32),
            mesh=plsc.VectorSubcoreMesh(core_axis_name="c", subcore_axis_name="s"),
            scratch_types=[
                pltpu.VMEM((_BUF,), jnp.float32),
                pltpu.VMEM((_OUT,), jnp.float32),
            ],
        )(_sc_select_body))
    return _sc_select_cache[0]


def kernel(query_embed, operators_embedding, Wq, bq, Wo, bo):
    wq_flat = Wq.reshape(_L * _H, _D)
    wo_flat = Wo.reshape(_L * _H, 2 * _D)
    buf = pl.pallas_call(
        _precompute_body,
        in_specs=[
            pl.BlockSpec(memory_space=pl.ANY),
            pl.BlockSpec(memory_space=pl.ANY),
            pl.BlockSpec(memory_space=pl.ANY),
            pl.BlockSpec((_L, _H), lambda: (0, 0)),
            pl.BlockSpec(memory_space=pl.ANY),
            pl.BlockSpec((_L, _H), lambda: (0, 0)),
        ],
        out_shape=jax.ShapeDtypeStruct((_BUF // _N, _N), jnp.float32),
        scratch_shapes=[
            pltpu.VMEM((1, _D), jnp.float32),
            pltpu.VMEM((_N, _D), jnp.float32),
            pltpu.VMEM((_L * _H, _D), jnp.float32),
            pltpu.VMEM((_L * _H, _D), jnp.float32),
            pltpu.VMEM((_L * _H, _D), jnp.float32),
            pltpu.SemaphoreType.DMA((_NC + 1, 4)),
        ],
    )(query_embed, operators_embedding, wq_flat, bq, wo_flat, bo)
    out = _get_sc_select()(buf.reshape(-1))
    probs = out[:_L * _N].reshape(_L, _N)
    logp = out[_L * _N:].reshape(_L, 16)[:, 0]
    return (logp, probs)


# speedup vs baseline: 1.0989x; 1.0989x over previous
import functools
import jax
import jax.numpy as jnp
from jax import lax
from jax.experimental import pallas as pl
from jax.experimental.pallas import tpu as pltpu
from jax.experimental.pallas import tpu_sc as plsc

_D = 4096
_H = 32
_L = 4
_N = 64
_THR = 0.3
_LN2 = 0.6931471805599453

_OFF_AQ = 0
_OFF_GQ = _L * _N
_OFF_SA2 = 2 * _L * _N
_OFF_SG2 = 3 * _L * _N
_OFF_MT = 4 * _L * _N
_BUF = _OFF_MT + (_L - 1) * _N * _N
_OUT = (_L + 1) * _N


def _precompute_body(q_ref, ops_ref, wq_ref, bq_ref, wo_ref, bo_ref, out_ref):
    qvec = q_ref[...]
    ops = ops_ref[...]
    wq = wq_ref[...]
    wo = wo_ref[...]
    bq_all = bq_ref[...]
    bo_all = bo_ref[...]

    dn = (((1,), (1,)), ((), ()))
    qproj = jax.lax.dot_general(qvec, wq, dn, preferred_element_type=jnp.float32)
    A = jax.lax.dot_general(ops, wo[:, :_D], dn, preferred_element_type=jnp.float32)
    G = jax.lax.dot_general(ops, wo[:, _D:], dn, preferred_element_type=jnp.float32)

    ones_h = jnp.ones((1, _H), dtype=jnp.float32)
    aq_rows, gq_rows, sa2_rows, sg2_rows, m_blocks = [], [], [], [], []
    for l in range(_L):
        qs = qproj[:, l * _H:(l + 1) * _H] + bq_all[l:l + 1, :]
        qn = qs / jnp.maximum(jnp.sqrt(jnp.sum(qs * qs)), 1e-12)
        Ap = A[:, l * _H:(l + 1) * _H] + bo_all[l:l + 1, :]
        Gl = G[:, l * _H:(l + 1) * _H]
        aq_rows.append(jax.lax.dot_general(qn, Ap, dn, preferred_element_type=jnp.float32))
        gq_rows.append(jax.lax.dot_general(qn, Gl, dn, preferred_element_type=jnp.float32))
        sa2_rows.append(jax.lax.dot_general(ones_h, Ap * Ap, dn, preferred_element_type=jnp.float32))
        sg2_rows.append(jax.lax.dot_general(ones_h, Gl * Gl, dn, preferred_element_type=jnp.float32))
        if l > 0:
            m_blocks.append(jax.lax.dot_general(Gl, Ap, dn, preferred_element_type=jnp.float32))

    out_ref[...] = jnp.concatenate(
        aq_rows + gq_rows + sa2_rows + sg2_rows + m_blocks, axis=0)


def _rsqrt_nr(x):
    bits = lax.bitcast_convert_type(x, jnp.int32)
    y = lax.bitcast_convert_type(jnp.int32(0x5F3759DF) - (bits >> 1), jnp.float32)
    for _ in range(3):
        y = y * (1.5 - 0.5 * x * y * y)
    return y


def _log_nr(x):
    bits = lax.bitcast_convert_type(x, jnp.int32)
    ex = ((bits >> 23) & jnp.int32(0xFF)) - 127
    mant = lax.bitcast_convert_type((bits & jnp.int32(0x007FFFFF)) | jnp.int32(0x3F800000),
                                    jnp.float32)
    y = ex.astype(jnp.float32) * _LN2 + (mant - 1.0)
    for _ in range(4):
        y = y + x * jnp.exp(-y) - 1.0
    return y


def _vgather(x, idx):
    dnums = lax.GatherDimensionNumbers(offset_dims=(), collapsed_slice_dims=(0,),
                                       start_index_map=(0,))
    return lax.gather(x, idx[:, None], dnums, slice_sizes=(1,),
                      mode=lax.GatherScatterMode.PROMISE_IN_BOUNDS)


def _sc_select_body(buf_hbm, out_hbm, buf_v, out_v):
    cid = lax.axis_index("c")
    sid = lax.axis_index("s")

    @pl.when(jnp.logical_and(cid == 0, sid == 0))
    def _():
        pltpu.sync_copy(buf_hbm, buf_v)

        iota = lax.iota(jnp.int32, 16)

        def red(v, op):
            for sh in (8, 4, 2, 1):
                v = op(v, _vgather(v, iota ^ sh))
            return v

        def add(a, b):
            return a + b

        onef = jnp.broadcast_to(jnp.float32(1.0), (16,))
        zerof = jnp.broadcast_to(jnp.float32(0.0), (16,))
        miss = jnp.broadcast_to(jnp.int32(192), (16,))
        fs = jnp.int32(0)
        for l in range(_L):
            sr = [buf_v[pl.ds(_OFF_AQ + l * _N + 16 * j, 16)] for j in range(4)]
            ss = [buf_v[pl.ds(_OFF_SA2 + l * _N + 16 * j, 16)] for j in range(4)]
            if l > 0:
                gq_f = jnp.broadcast_to(buf_v[pl.ds(_OFF_GQ + l * _N + fs, 16)][0], (16,))
                sg_f = jnp.broadcast_to(buf_v[pl.ds(_OFF_SG2 + l * _N + fs, 16)][0], (16,))
                mbase = _OFF_MT + (l - 1) * _N * _N + fs * _N
                for j in range(4):
                    mcol = buf_v[pl.ds(mbase + 16 * j, 16)]
                    ss[j] = ss[j] + 2.0 * mcol + sg_f
                    sr[j] = sr[j] + gq_f
            sc = [sr[j] * _rsqrt_nr(jnp.maximum(ss[j], 1e-24)) for j in range(4)]
            mx = red(jnp.maximum(jnp.maximum(sc[0], sc[1]),
                                 jnp.maximum(sc[2], sc[3])), jnp.maximum)
            e = [jnp.exp(sc[j] - mx) for j in range(4)]
            sv = red(e[0] + e[1] + e[2] + e[3], add)
            p = [e[j] / sv for j in range(4)]
            logs = _log_nr(sv)
            lp = [sc[j] - mx - logs for j in range(4)]
            maskb = [p[j] > _THR for j in range(4)]
            maskf = [jnp.where(maskb[j], onef, zerof) for j in range(4)]
            pmax = red(jnp.maximum(jnp.maximum(p[0], p[1]),
                                   jnp.maximum(p[2], p[3])), jnp.maximum)
            key = red(jnp.minimum(
                jnp.minimum(
                    jnp.where(maskb[0], iota,
                              jnp.where(p[0] == pmax, iota + 64, miss)),
                    jnp.where(maskb[1], iota + 16,
                              jnp.where(p[1] == pmax, iota + 80, miss))),
                jnp.minimum(
                    jnp.where(maskb[2], iota + 32,
                              jnp.where(p[2] == pmax, iota + 96, miss)),
                    jnp.where(maskb[3], iota + 48,
                              jnp.where(p[3] == pmax, iota + 112, miss)))),
                jnp.minimum)
            has_any = key < _N
            fidx = key & 63
            sel = [jnp.where(has_any, maskf[j],
                             jnp.where((iota + 16 * j) == fidx, onef, zerof))
                   for j in range(4)]
            llp = red(sel[0] * lp[0] + sel[1] * lp[1] + sel[2] * lp[2] + sel[3] * lp[3],
                      add)
            fs = fidx[0]
            for j in range(4):
                out_v[pl.ds(l * _N + 16 * j, 16)] = p[j]
            out_v[pl.ds(_L * _N + l * 16, 16)] = llp

        pltpu.sync_copy(out_v, out_hbm)


_sc_select_cache = []


def _get_sc_select():
    if not _sc_select_cache:
        _sc_select_cache.append(functools.partial(
            pl.kernel,
            out_type=jax.ShapeDtypeStruct((_OUT,), jnp.float32),
            mesh=plsc.VectorSubcoreMesh(core_axis_name="c", subcore_axis_name="s", num_cores=1),
            scratch_types=[
                pltpu.VMEM((_BUF,), jnp.float32),
                pltpu.VMEM((_OUT,), jnp.float32),
            ],
        )(_sc_select_body))
    return _sc_select_cache[0]


def kernel(query_embed, operators_embedding, Wq, bq, Wo, bo):
    wq_flat = Wq.reshape(_L * _H, _D)
    wo_flat = Wo.reshape(_L * _H, 2 * _D)
    buf = pl.pallas_call(
        _precompute_body,
        out_shape=jax.ShapeDtypeStruct((_BUF // _N, _N), jnp.float32),
    )(query_embed, operators_embedding, wq_flat, bq, wo_flat, bo)
    out = _get_sc_select()(buf.reshape(-1))
    probs = out[:_L * _N].reshape(_L, _N)
    logp = out[_L * _N:].reshape(_L, 16)[:, 0]
    return (logp, probs)
